# baseline (speedup 1.0000x reference)
import functools

import jax
import jax.numpy as jnp
from jax import lax
from jax.experimental import pallas as pl
from jax.experimental.pallas import tpu as pltpu
from jax.experimental.pallas import tpu_sc as plsc

N_FACTORS = 64
HIDDEN = 256
BATCH = 16384
MAX_RATING = 5.0
MIN_RATING = 0.5

NC = 2
NS = 16
NW = NC * NS
B_PER_W = BATCH // NW
IDX_CHUNK = 128
N_CHUNKS = B_PER_W // IDX_CHUNK


def _sc_gather(user2d, movie2d, U, M):
    mesh = plsc.VectorSubcoreMesh(core_axis_name="c", subcore_axis_name="s")

    @functools.partial(
        pl.kernel,
        out_type=(
            jax.ShapeDtypeStruct((BATCH, N_FACTORS), jnp.float32),
            jax.ShapeDtypeStruct((BATCH, N_FACTORS), jnp.float32),
        ),
        mesh=mesh,
        scratch_types=[
            pltpu.VMEM((N_CHUNKS, IDX_CHUNK), jnp.int32),
            pltpu.VMEM((N_CHUNKS, IDX_CHUNK), jnp.int32),
            pltpu.VMEM((B_PER_W, N_FACTORS), jnp.float32),
            pltpu.VMEM((B_PER_W, N_FACTORS), jnp.float32),
            pltpu.SemaphoreType.DMA,
        ],
        compiler_params=pltpu.CompilerParams(use_tc_tiling_on_sc=False),
    )
    def k(user_hbm, movie_hbm, u_hbm, m_hbm, ue_hbm, me_hbm,
          uidx_v, midx_v, urows_v, mrows_v, sem):
        wid = lax.axis_index("s") * NC + lax.axis_index("c")
        base = wid * B_PER_W
        crow = wid * N_CHUNKS
        pltpu.sync_copy(user_hbm.at[pl.ds(crow, N_CHUNKS)], uidx_v)
        pltpu.sync_copy(movie_hbm.at[pl.ds(crow, N_CHUNKS)], midx_v)
        copies = []
        for j in range(N_CHUNKS):
            dst_u = urows_v.at[pl.ds(j * IDX_CHUNK, IDX_CHUNK)]
            dst_m = mrows_v.at[pl.ds(j * IDX_CHUNK, IDX_CHUNK)]
            cu = pltpu.make_async_copy(u_hbm.at[uidx_v.at[j]], dst_u, sem)
            cm = pltpu.make_async_copy(m_hbm.at[midx_v.at[j]], dst_m, sem)
            cu.start()
            cm.start()
            copies.append(cu)
            copies.append(cm)
        for c in copies:
            c.wait()
        pltpu.sync_copy(urows_v, ue_hbm.at[pl.ds(base, B_PER_W)])
        pltpu.sync_copy(mrows_v, me_hbm.at[pl.ds(base, B_PER_W)])

    return k(user2d, movie2d, U, M)


def _mlp_body(ue_ref, me_ref, w1_ref, b1_ref, w2_ref, b2_ref, out_ref):
    ue = ue_ref[...]
    me = me_ref[...]
    w1 = w1_ref[...]
    h = lax.dot_general(ue, w1[:, :N_FACTORS],
                        (((1,), (1,)), ((), ())),
                        preferred_element_type=jnp.float32)
    h = h + lax.dot_general(me, w1[:, N_FACTORS:],
                            (((1,), (1,)), ((), ())),
                            preferred_element_type=jnp.float32)
    h = jnp.maximum(h + b1_ref[0, :][None, :], 0.0)
    y = jnp.sum(h * w2_ref[0, :][None, :], axis=1, keepdims=True)
    y = y + b2_ref[0, 0]
    out_ref[...] = jax.nn.sigmoid(y) * (MAX_RATING - MIN_RATING + 1.0) + (
        MIN_RATING - 0.5)


def _tc_mlp(ue, me, W1, b1, W2, b2, blk=2048):
    grid = (BATCH // blk,)
    return pl.pallas_call(
        _mlp_body,
        grid=grid,
        in_specs=[
            pl.BlockSpec((blk, N_FACTORS), lambda i: (i, 0)),
            pl.BlockSpec((blk, N_FACTORS), lambda i: (i, 0)),
            pl.BlockSpec((HIDDEN, 2 * N_FACTORS), lambda i: (0, 0)),
            pl.BlockSpec((1, HIDDEN), lambda i: (0, 0)),
            pl.BlockSpec((1, HIDDEN), lambda i: (0, 0)),
            pl.BlockSpec((1, 1), lambda i: (0, 0), memory_space=pltpu.SMEM),
        ],
        out_specs=pl.BlockSpec((blk, 1), lambda i: (i, 0)),
        out_shape=jax.ShapeDtypeStruct((BATCH, 1), jnp.float32),
    )(ue, me, W1, b1, W2, b2)


@jax.jit
def kernel(user, movie, U, M, W1, b1, W2, b2):
    user2d = user.astype(jnp.int32).reshape(BATCH // IDX_CHUNK, IDX_CHUNK)
    movie2d = movie.astype(jnp.int32).reshape(BATCH // IDX_CHUNK, IDX_CHUNK)
    ue, me = _sc_gather(user2d, movie2d, U, M)
    return _tc_mlp(ue, me, W1, b1.reshape(1, HIDDEN), W2, b2.reshape(1, 1))
